# baseline (speedup 1.0000x reference)
import functools
import math

import jax
import jax.numpy as jnp
from jax import lax
from jax.experimental import pallas as pl
from jax.experimental.pallas import tpu as pltpu
from jax.experimental.pallas import tpu_sc as plsc

EMB = 64
SCALE = math.sqrt(EMB)
LANES = 16

_BLK = 256
_S = 200
_BDIM = 4096
_NBB = _BDIM // 128
_NJB = _BDIM // _BLK
_NB = 4


@functools.cache
def _build():
    info = plsc.get_sparse_core_info()
    NC, NS = info.num_cores, info.num_subcores
    NW = NC * NS
    n_jobs = _S * _NJB
    jobs_w = n_jobs // NW
    assert jobs_w % _NB == 0 and jobs_w >= 3 * _NB
    toks_w = jobs_w * _BLK
    mesh = plsc.VectorSubcoreMesh(core_axis_name="c", subcore_axis_name="s")

    @functools.partial(
        pl.kernel,
        mesh=mesh,
        out_type=jax.ShapeDtypeStruct((_S, EMB // 8, _NBB, 8, 128), jnp.float32),
        scratch_types=[
            pltpu.VMEM((toks_w,), jnp.int32),
            pltpu.VMEM((_NB, _BLK), jnp.int32),
            pltpu.VMEM((_NB, _BLK, EMB), jnp.float32),
            pltpu.VMEM((2, EMB // 8, 2, 8, 128), jnp.float32),
            pltpu.SemaphoreType.DMA,
            pltpu.SemaphoreType.DMA,
            pltpu.SemaphoreType.DMA,
            pltpu.SemaphoreType.DMA,
            pltpu.SemaphoreType.DMA,
            pltpu.SemaphoreType.DMA,
        ],
        compiler_params=pltpu.CompilerParams(
            use_tc_tiling_on_sc=False, needs_layout_passes=False
        ),
    )
    def k(tok_hbm, table_hbm, out_hbm, tk_v, idx_v, g_v, t_v,
          g0, g1, g2, g3, o0, o1):
        wid = lax.axis_index("s") * NC + lax.axis_index("c")
        job0 = wid * jobs_w
        gsem = (g0, g1, g2, g3)
        osem = (o0, o1)
        lane = lax.iota(jnp.int32, LANES)

        pltpu.sync_copy(tok_hbm.at[pl.ds(wid * toks_w, toks_w)], tk_v)

        def issue(j, nb):
            for q in range(_BLK // LANES):
                sl = pl.ds(j * _BLK + q * LANES, LANES)
                idx_v[nb, pl.ds(q * LANES, LANES)] = tk_v[sl] * 2
            pltpu.async_copy(table_hbm.at[idx_v.at[nb]], g_v.at[nb], gsem[nb])

        def finish(j, nb, tb, first):
            jid = job0 + j
            s = lax.shift_right_logical(jid, 4)
            bb2 = lax.bitwise_and(jid, _NJB - 1)
            pltpu.make_async_copy(
                table_hbm.at[idx_v.at[nb]], g_v.at[nb], gsem[nb]
            ).wait()
            if not first:
                pltpu.make_async_copy(
                    t_v.at[tb], out_hbm.at[0, :, pl.ds(0, 2)], osem[tb]
                ).wait()

            @plsc.parallel_loop(0, EMB, unroll=4)
            def _tr(e):
                eb = lax.shift_right_logical(e, 3)
                ei = lax.bitwise_and(e, 7)
                col = jnp.full((LANES,), 0, jnp.int32) + e
                for q in range(_BLK // LANES):
                    row = lane + (q * LANES)
                    v = plsc.load_gather(g_v.at[nb], [row, col])
                    t_v[tb, eb, q // 8, ei,
                        pl.ds((q % 8) * LANES, LANES)] = v * SCALE

            pltpu.async_copy(
                t_v.at[tb], out_hbm.at[s, :, pl.ds(bb2 * 2, 2)], osem[tb]
            )

        def quad(a, first):
            issue(a + 3, 3)
            finish(a + 0, 0, 0, first)
            issue(a + 4, 0)
            finish(a + 1, 1, 1, first)
            issue(a + 5, 1)
            finish(a + 2, 2, 0, False)
            issue(a + 6, 2)
            finish(a + 3, 3, 1, False)

        issue(0, 0)
        issue(1, 1)
        issue(2, 2)
        quad(0, True)

        def body(gg, carry):
            quad(4 * gg, False)
            return carry

        lax.fori_loop(1, jobs_w // 4 - 1, body, 0)

        a = jobs_w - 4
        issue(a + 3, 3)
        finish(a + 0, 0, 0, False)
        finish(a + 1, 1, 1, False)
        finish(a + 2, 2, 0, False)
        finish(a + 3, 3, 1, False)
        for tb in range(2):
            pltpu.make_async_copy(
                t_v.at[tb], out_hbm.at[0, :, pl.ds(0, 2)], osem[tb]
            ).wait()

    return k


def kernel(tokens, table):
    tok_flat = tokens.T.astype(jnp.int32).reshape(-1)
    table_pad = jnp.pad(table, ((0, 0), (0, EMB)))
    table_2m = table_pad.reshape(2 * table.shape[0], EMB)
    out5 = _build()(tok_flat, table_2m)
    return out5.transpose(2, 4, 0, 1, 3).reshape(_BDIM, _S, EMB)


# speedup vs baseline: 1.0065x; 1.0065x over previous
import functools
import math

import jax
import jax.numpy as jnp
from jax import lax
from jax.experimental import pallas as pl
from jax.experimental.pallas import tpu as pltpu
from jax.experimental.pallas import tpu_sc as plsc

EMB = 64
SCALE = math.sqrt(EMB)
LANES = 16

_BLK = 128
_S = 200
_BDIM = 4096
_NBB = _BDIM // _BLK
_NB = 4


@functools.cache
def _build():
    info = plsc.get_sparse_core_info()
    NC, NS = info.num_cores, info.num_subcores
    NW = NC * NS
    n_jobs = _S * _NBB
    jobs_w = n_jobs // NW
    assert jobs_w % _NB == 0 and jobs_w >= 3 * _NB
    toks_w = jobs_w * _BLK
    mesh = plsc.VectorSubcoreMesh(core_axis_name="c", subcore_axis_name="s")

    @functools.partial(
        pl.kernel,
        mesh=mesh,
        out_type=jax.ShapeDtypeStruct((_S, EMB // 8, _NBB, 8, _BLK), jnp.float32),
        scratch_types=[
            pltpu.VMEM((toks_w,), jnp.int32),
            pltpu.VMEM((_NB, _BLK), jnp.int32),
            pltpu.VMEM((_NB, _BLK, EMB), jnp.float32),
            pltpu.VMEM((_NB, EMB // 8, 8, _BLK), jnp.float32),
            pltpu.SemaphoreType.DMA,
            pltpu.SemaphoreType.DMA,
            pltpu.SemaphoreType.DMA,
            pltpu.SemaphoreType.DMA,
            pltpu.SemaphoreType.DMA,
            pltpu.SemaphoreType.DMA,
            pltpu.SemaphoreType.DMA,
            pltpu.SemaphoreType.DMA,
        ],
        compiler_params=pltpu.CompilerParams(
            use_tc_tiling_on_sc=False, needs_layout_passes=False
        ),
    )
    def k(tok_hbm, table_hbm, out_hbm, tk_v, idx_v, g_v, t_v,
          g0, g1, g2, g3, o0, o1, o2, o3):
        wid = lax.axis_index("s") * NC + lax.axis_index("c")
        job0 = wid * jobs_w
        gsem = (g0, g1, g2, g3)
        osem = (o0, o1, o2, o3)
        lane = lax.iota(jnp.int32, LANES)

        pltpu.sync_copy(tok_hbm.at[pl.ds(wid * toks_w, toks_w)], tk_v)

        def issue(j, nb):
            for q in range(_BLK // LANES):
                sl = pl.ds(j * _BLK + q * LANES, LANES)
                idx_v[nb, pl.ds(q * LANES, LANES)] = tk_v[sl] * 2
            pltpu.async_copy(table_hbm.at[idx_v.at[nb]], g_v.at[nb], gsem[nb])

        def finish(j, nb, first):
            jid = job0 + j
            s = lax.shift_right_logical(jid, 5)
            bb = lax.bitwise_and(jid, _NBB - 1)
            pltpu.make_async_copy(
                table_hbm.at[idx_v.at[nb]], g_v.at[nb], gsem[nb]
            ).wait()
            if not first:
                pltpu.make_async_copy(
                    t_v.at[nb], out_hbm.at[0, :, 0], osem[nb]
                ).wait()

            @plsc.parallel_loop(0, EMB, unroll=4)
            def _tr(e):
                eb = lax.shift_right_logical(e, 3)
                ei = lax.bitwise_and(e, 7)
                col = jnp.full((LANES,), 0, jnp.int32) + e
                for q in range(_BLK // LANES):
                    row = lane + (q * LANES)
                    v = plsc.load_gather(g_v.at[nb], [row, col])
                    t_v[nb, eb, ei, pl.ds(q * LANES, LANES)] = v * SCALE

            pltpu.async_copy(t_v.at[nb], out_hbm.at[s, :, bb], osem[nb])

        def quad(a, first):
            issue(a + 3, 3)
            finish(a + 0, 0, first)
            issue(a + 4, 0)
            finish(a + 1, 1, first)
            issue(a + 5, 1)
            finish(a + 2, 2, first)
            issue(a + 6, 2)
            finish(a + 3, 3, first)

        issue(0, 0)
        issue(1, 1)
        issue(2, 2)
        quad(0, True)

        def body(gg, carry):
            quad(4 * gg, False)
            return carry

        lax.fori_loop(1, jobs_w // 4 - 1, body, 0)

        a = jobs_w - 4
        issue(a + 3, 3)
        finish(a + 0, 0, False)
        finish(a + 1, 1, False)
        finish(a + 2, 2, False)
        finish(a + 3, 3, False)
        for nb in range(_NB):
            pltpu.make_async_copy(
                t_v.at[nb], out_hbm.at[0, :, 0], osem[nb]
            ).wait()

    return k


def kernel(tokens, table):
    tok_flat = tokens.T.astype(jnp.int32).reshape(-1)
    table_pad = jnp.pad(table, ((0, 0), (0, EMB)))
    table_2m = table_pad.reshape(2 * table.shape[0], EMB)
    out5 = _build()(tok_flat, table_2m)
    return out5.transpose(2, 4, 0, 1, 3).reshape(_BDIM, _S, EMB)
